# Initial kernel scaffold; baseline (speedup 1.0000x reference)
import jax
import jax.numpy as jnp
from jax.experimental import pallas as pl


def kernel(x, wg, fc1_w, fc1_b, fc2_w, fc2_b):
    raise NotImplementedError("write your pallas kernel here")


# speedup vs baseline: 1.1086x; 1.1086x over previous
import functools

import jax
import jax.numpy as jnp
from jax import lax
from jax.experimental import pallas as pl
from jax.experimental.pallas import tpu as pltpu
from jax.experimental.pallas import tpu_sc as plsc

E = 16
K = 2
D = 1024
H = 2048
T = 4096
CAP = 640

_NW = 32
_ROWS = E * CAP
_RPT = _ROWS // _NW
_RCHUNK = 32
_TPT = T // _NW
_TCHUNK = 32


def _cumsum0(a):
    s = 1
    n = a.shape[0]
    while s < n:
        pad = jnp.zeros((s, a.shape[1]), a.dtype)
        a = a + jnp.concatenate([pad, a[:-s, :]], axis=0)
        s *= 2
    return a


def _router_body(x_ref, wg_ref, mi_ref, mf_ref):
    x = x_ref[...]
    wg = wg_ref[...]
    logits = jnp.dot(x, wg, preferred_element_type=jnp.float32)
    m = jnp.max(logits, axis=1, keepdims=True)
    ex = jnp.exp(logits - m)
    gates = ex / jnp.sum(ex, axis=1, keepdims=True)

    lane = lax.broadcasted_iota(jnp.int32, (T, E), 1)
    v0 = jnp.max(gates, axis=1, keepdims=True)
    i0 = jnp.min(jnp.where(gates == v0, lane, E), axis=1, keepdims=True)
    g1 = jnp.where(lane == i0, -1.0, gates)
    v1 = jnp.max(g1, axis=1, keepdims=True)
    i1 = jnp.min(jnp.where(g1 == v1, lane, E), axis=1, keepdims=True)

    denom = v0 + v1 + 1e-9
    w0 = v0 / denom
    w1 = v1 / denom

    oh0 = (lane == i0).astype(jnp.float32)
    oh1 = (lane == i1).astype(jnp.float32)
    c0 = _cumsum0(oh0)
    c1 = _cumsum0(oh1) + c0[T - 1:T, :]
    p0 = jnp.sum(oh0 * c0, axis=1, keepdims=True) - 1.0
    p1 = jnp.sum(oh1 * c1, axis=1, keepdims=True) - 1.0
    p0i = p0.astype(jnp.int32)
    p1i = p1.astype(jnp.int32)

    w0 = w0 * (p0i < CAP).astype(jnp.float32)
    w1 = w1 * (p1i < CAP).astype(jnp.float32)

    zi = jnp.zeros((T, E - 4), jnp.int32)
    zf = jnp.zeros((T, E - 2), jnp.float32)
    mi_ref[...] = jnp.concatenate([i0, i1, p0i, p1i, zi], axis=1)
    mf_ref[...] = jnp.concatenate([w0, w1, zf], axis=1)


def _router(x, wg):
    return pl.pallas_call(
        _router_body,
        out_shape=[
            jax.ShapeDtypeStruct((T, E), jnp.int32),
            jax.ShapeDtypeStruct((T, E), jnp.float32),
        ],
    )(x, wg)


def _dispatch(x_ext, mi):
    mesh = plsc.VectorSubcoreMesh(core_axis_name="c", subcore_axis_name="s")

    @functools.partial(
        pl.kernel,
        mesh=mesh,
        compiler_params=pltpu.CompilerParams(needs_layout_passes=False),
        out_type=jax.ShapeDtypeStruct((_ROWS, D), jnp.float32),
        scratch_types=[
            pltpu.VMEM((T * E,), jnp.int32),
            pltpu.VMEM((_ROWS,), jnp.int32),
            pltpu.VMEM((_RCHUNK, D), jnp.float32),
            pltpu.SemaphoreType.DMA,
        ],
    )
    def k(x_hbm, mi_hbm, disp_hbm, meta_vm, inv_vm, rows_vm, sem):
        wid = lax.axis_index("s") * 2 + lax.axis_index("c")
        pltpu.sync_copy(mi_hbm, meta_vm)

        zero16 = jnp.zeros((16,), jnp.int32)

        def zbody(i, carry):
            inv_vm[pl.ds(i * 16, 16)] = zero16
            return carry

        lax.fori_loop(0, _ROWS // 16, zbody, 0)

        lanes = lax.iota(jnp.int32, 16)
        col0 = jnp.zeros((16,), jnp.int32)

        def sbody(kk, carry):
            t16 = kk * 16 + lanes
            f16 = t16 * E
            i0 = plsc.load_gather(meta_vm, [f16 + col0])
            i1 = plsc.load_gather(meta_vm, [f16 + (col0 + 1)])
            p0 = plsc.load_gather(meta_vm, [f16 + (col0 + 2)])
            p1 = plsc.load_gather(meta_vm, [f16 + (col0 + 3)])
            keep0 = p0 < CAP
            keep1 = p1 < CAP
            d0 = jnp.where(keep0, i0 * CAP + p0, 0)
            d1 = jnp.where(keep1, i1 * CAP + p1, 0)
            plsc.store_scatter(inv_vm, [d0], t16 + 1, mask=keep0)
            plsc.store_scatter(inv_vm, [d1], t16 + 1, mask=keep1)
            return carry

        lax.fori_loop(0, T // 16, sbody, 0)

        base = wid * _RPT

        def gbody(c, carry):
            rb = base + c * _RCHUNK
            idx = inv_vm.at[pl.ds(rb, _RCHUNK)]
            pltpu.async_copy(x_hbm.at[idx], rows_vm, sem).wait()
            pltpu.sync_copy(rows_vm, disp_hbm.at[pl.ds(rb, _RCHUNK)])
            return carry

        lax.fori_loop(0, _RPT // _RCHUNK, gbody, 0)

    return k(x_ext, mi)


def _ffn_body(d_ref, w1_ref, b1_ref, w2_ref, b2_ref, y_ref):
    d = d_ref[...].astype(jnp.bfloat16)
    w1 = w1_ref[0].astype(jnp.bfloat16)
    h = lax.dot_general(d, w1, (((1,), (1,)), ((), ())),
                        preferred_element_type=jnp.float32)
    h = jnp.maximum(h + b1_ref[0], 0.0)
    w2 = w2_ref[0].astype(jnp.bfloat16)
    y = lax.dot_general(h.astype(jnp.bfloat16), w2,
                        (((1,), (0,)), ((), ())),
                        preferred_element_type=jnp.float32)
    y_ref[...] = y + b2_ref[0]


def _ffn(disp, fc1_w, fc1_b, fc2_w, fc2_b):
    return pl.pallas_call(
        _ffn_body,
        grid=(E,),
        in_specs=[
            pl.BlockSpec((CAP, D), lambda e: (e, 0)),
            pl.BlockSpec((1, H, D), lambda e: (e, 0, 0)),
            pl.BlockSpec((1, 1, H), lambda e: (e, 0, 0)),
            pl.BlockSpec((1, H, D), lambda e: (e, 0, 0)),
            pl.BlockSpec((1, 1, D), lambda e: (e, 0, 0)),
        ],
        out_specs=pl.BlockSpec((CAP, D), lambda e: (e, 0)),
        out_shape=jax.ShapeDtypeStruct((_ROWS, D), jnp.float32),
    )(disp, fc1_w, fc1_b.reshape(E, 1, H), fc2_w, fc2_b.reshape(E, 1, D))


def _combine(y, mi, mf):
    mesh = plsc.VectorSubcoreMesh(core_axis_name="c", subcore_axis_name="s")

    @functools.partial(
        pl.kernel,
        mesh=mesh,
        compiler_params=pltpu.CompilerParams(needs_layout_passes=False),
        out_type=jax.ShapeDtypeStruct((T, D), jnp.float32),
        scratch_types=[
            pltpu.VMEM((_TPT * E,), jnp.int32),
            pltpu.VMEM((_TPT * E,), jnp.float32),
            pltpu.VMEM((_TCHUNK,), jnp.int32),
            pltpu.VMEM((_TCHUNK,), jnp.int32),
            pltpu.VMEM((_TCHUNK, D), jnp.float32),
            pltpu.VMEM((_TCHUNK, D), jnp.float32),
            pltpu.VMEM((_TCHUNK, D), jnp.float32),
            pltpu.SemaphoreType.DMA,
        ],
    )
    def k(y_hbm, mi_hbm, mf_hbm, out_hbm,
          mi_vm, mf_vm, d0_vm, d1_vm, r0_vm, r1_vm, o_vm, sem):
        wid = lax.axis_index("s") * 2 + lax.axis_index("c")
        tbase = wid * _TPT
        pltpu.sync_copy(mi_hbm.at[pl.ds(tbase * E, _TPT * E)], mi_vm)
        pltpu.sync_copy(mf_hbm.at[pl.ds(tbase * E, _TPT * E)], mf_vm)

        lanes = lax.iota(jnp.int32, 16)
        col0 = jnp.zeros((16,), jnp.int32)

        def chunk(ci, carry):
            def dbody(q, carry2):
                tloc = (ci * _TCHUNK + q * 16 + lanes) * E
                i0 = plsc.load_gather(mi_vm, [tloc + col0])
                i1 = plsc.load_gather(mi_vm, [tloc + (col0 + 1)])
                p0 = plsc.load_gather(mi_vm, [tloc + (col0 + 2)])
                p1 = plsc.load_gather(mi_vm, [tloc + (col0 + 3)])
                d0_vm[pl.ds(q * 16, 16)] = i0 * CAP + jnp.minimum(p0, CAP - 1)
                d1_vm[pl.ds(q * 16, 16)] = i1 * CAP + jnp.minimum(p1, CAP - 1)
                return carry2

            lax.fori_loop(0, _TCHUNK // 16, dbody, 0)
            pltpu.async_copy(y_hbm.at[d0_vm], r0_vm, sem).wait()
            pltpu.async_copy(y_hbm.at[d1_vm], r1_vm, sem).wait()

            def rbody(r, carry2):
                tl = col0 + (ci * _TCHUNK + r) * E
                w0 = plsc.load_gather(mf_vm, [tl])
                w1 = plsc.load_gather(mf_vm, [tl + 1])

                def cbody(cc, carry3):
                    a = r0_vm[r, pl.ds(cc * 16, 16)]
                    b = r1_vm[r, pl.ds(cc * 16, 16)]
                    o_vm[r, pl.ds(cc * 16, 16)] = w0 * a + w1 * b
                    return carry3

                lax.fori_loop(0, D // 16, cbody, 0)
                return carry2

            lax.fori_loop(0, _TCHUNK, rbody, 0)
            pltpu.sync_copy(o_vm, out_hbm.at[pl.ds(tbase + ci * _TCHUNK, _TCHUNK)])
            return carry

        lax.fori_loop(0, _TPT // _TCHUNK, chunk, 0)

    return k(y, mi, mf)


def kernel(x, wg, fc1_w, fc1_b, fc2_w, fc2_b):
    mi, mf = _router(x, wg)
    mi_f = mi.reshape(T * E)
    mf_f = mf.reshape(T * E)
    x_ext = jnp.concatenate([jnp.zeros((1, D), x.dtype), x], axis=0)
    disp = _dispatch(x_ext, mi_f)
    y = _ffn(disp, fc1_w, fc1_b, fc2_w, fc2_b)
    return _combine(y, mi_f, mf_f)


# speedup vs baseline: 1.1232x; 1.0131x over previous
import functools

import jax
import jax.numpy as jnp
from jax import lax
from jax.experimental import pallas as pl
from jax.experimental.pallas import tpu as pltpu
from jax.experimental.pallas import tpu_sc as plsc

E = 16
K = 2
D = 1024
H = 2048
T = 4096
CAP = 640

_PBITS = 13
_PMASK = (1 << _PBITS) - 1

_NW = 32
_ROWS = E * CAP
_RPT = _ROWS // _NW
_RCHUNK = 32
_NRC = _RPT // _RCHUNK
_TPT = T // _NW
_TCHUNK = 16
_NTC = _TPT // _TCHUNK


def _cumsum0(a):
    s = 1
    n = a.shape[0]
    while s < n:
        pad = jnp.zeros((s, a.shape[1]), a.dtype)
        a = a + jnp.concatenate([pad, a[:-s, :]], axis=0)
        s *= 2
    return a


def _router_body(x_ref, wg_ref, code_ref, wgt_ref):
    x = x_ref[...]
    wg = wg_ref[...]
    logits = jnp.dot(x, wg, preferred_element_type=jnp.float32)
    m = jnp.max(logits, axis=1, keepdims=True)
    ex = jnp.exp(logits - m)
    gates = ex / jnp.sum(ex, axis=1, keepdims=True)

    lane = lax.broadcasted_iota(jnp.int32, (T, E), 1)
    v0 = jnp.max(gates, axis=1, keepdims=True)
    i0 = jnp.min(jnp.where(gates == v0, lane, E), axis=1, keepdims=True)
    g1 = jnp.where(lane == i0, -1.0, gates)
    v1 = jnp.max(g1, axis=1, keepdims=True)
    i1 = jnp.min(jnp.where(g1 == v1, lane, E), axis=1, keepdims=True)

    denom = v0 + v1 + 1e-9
    w0 = v0 / denom
    w1 = v1 / denom

    oh0 = (lane == i0).astype(jnp.float32)
    oh1 = (lane == i1).astype(jnp.float32)
    c0 = _cumsum0(oh0)
    c1 = _cumsum0(oh1) + c0[T - 1:T, :]
    p0 = jnp.sum(oh0 * c0, axis=1, keepdims=True) - 1.0
    p1 = jnp.sum(oh1 * c1, axis=1, keepdims=True) - 1.0
    p0i = p0.astype(jnp.int32)
    p1i = p1.astype(jnp.int32)

    w0 = w0 * (p0i < CAP).astype(jnp.float32)
    w1 = w1 * (p1i < CAP).astype(jnp.float32)

    code_ref[...] = jnp.concatenate(
        [i0 * (1 << _PBITS) + p0i, i1 * (1 << _PBITS) + p1i], axis=1)
    wgt_ref[...] = jnp.concatenate([w0, w1], axis=1)


def _router(x, wg):
    return pl.pallas_call(
        _router_body,
        out_shape=[
            jax.ShapeDtypeStruct((T, K), jnp.int32),
            jax.ShapeDtypeStruct((T, K), jnp.float32),
        ],
    )(x, wg)


def _dispatch(x_ext, code_f):
    mesh = plsc.VectorSubcoreMesh(core_axis_name="c", subcore_axis_name="s")

    @functools.partial(
        pl.kernel,
        mesh=mesh,
        compiler_params=pltpu.CompilerParams(needs_layout_passes=False),
        out_type=jax.ShapeDtypeStruct((_ROWS, D), jnp.float32),
        scratch_types=[
            pltpu.VMEM((T * K,), jnp.int32),
            pltpu.VMEM((_ROWS,), jnp.int32),
            pltpu.VMEM((_RCHUNK, D), jnp.float32),
            pltpu.VMEM((_RCHUNK, D), jnp.float32),
            pltpu.SemaphoreType.DMA,
            pltpu.SemaphoreType.DMA,
            pltpu.SemaphoreType.DMA,
            pltpu.SemaphoreType.DMA,
        ],
    )
    def k(x_hbm, code_hbm, disp_hbm, code_vm, inv_vm, rows0_vm, rows1_vm,
          g0_sem, g1_sem, w0_sem, w1_sem):
        wid = lax.axis_index("s") * 2 + lax.axis_index("c")
        pltpu.sync_copy(code_hbm, code_vm)

        zero16 = jnp.zeros((16,), jnp.int32)

        def zbody(i, carry):
            inv_vm[pl.ds(i * 16, 16)] = zero16
            return carry

        lax.fori_loop(0, _ROWS // 16, zbody, 0, unroll=8)

        lanes = lax.iota(jnp.int32, 16)

        def sbody(kk, carry):
            t16 = kk * 16 + lanes
            f16 = t16 * K
            c0 = plsc.load_gather(code_vm, [f16])
            c1 = plsc.load_gather(code_vm, [f16 + 1])
            i0 = lax.shift_right_logical(c0, _PBITS)
            p0 = jnp.bitwise_and(c0, _PMASK)
            i1 = lax.shift_right_logical(c1, _PBITS)
            p1 = jnp.bitwise_and(c1, _PMASK)
            keep0 = p0 < CAP
            keep1 = p1 < CAP
            d0 = jnp.where(keep0, i0 * CAP + p0, 0)
            d1 = jnp.where(keep1, i1 * CAP + p1, 0)
            plsc.store_scatter(inv_vm, [d0], t16 + 1, mask=keep0)
            plsc.store_scatter(inv_vm, [d1], t16 + 1, mask=keep1)
            return carry

        lax.fori_loop(0, T // 16, sbody, 0, unroll=4)

        base = wid * _RPT
        gbufs = (rows0_vm, rows1_vm)
        gsems = (g0_sem, g1_sem)
        wsems = (w0_sem, w1_sem)
        gwaits = [None, None]
        wwaits = [None, None]
        for c in range(_NRC + 1):
            b = c % 2
            if c < _NRC:
                if wwaits[b] is not None:
                    wwaits[b].wait()
                    wwaits[b] = None
                idx = inv_vm.at[pl.ds(base + c * _RCHUNK, _RCHUNK)]
                gwaits[b] = pltpu.async_copy(x_hbm.at[idx], gbufs[b], gsems[b])
            if c >= 1:
                pb = (c - 1) % 2
                gwaits[pb].wait()
                dst = disp_hbm.at[pl.ds(base + (c - 1) * _RCHUNK, _RCHUNK)]
                wwaits[pb] = pltpu.async_copy(gbufs[pb], dst, wsems[pb])
        for w in wwaits:
            if w is not None:
                w.wait()

    return k(x_ext, code_f)


def _ffn_body(d_ref, w1_ref, b1_ref, w2_ref, b2_ref, y_ref):
    d = d_ref[...].astype(jnp.bfloat16)
    w1 = w1_ref[0].astype(jnp.bfloat16)
    h = lax.dot_general(d, w1, (((1,), (1,)), ((), ())),
                        preferred_element_type=jnp.float32)
    h = jnp.maximum(h + b1_ref[0], 0.0)
    w2 = w2_ref[0].astype(jnp.bfloat16)
    y = lax.dot_general(h.astype(jnp.bfloat16), w2,
                        (((1,), (0,)), ((), ())),
                        preferred_element_type=jnp.float32)
    y_ref[...] = y + b2_ref[0]


def _ffn(disp, fc1_w, fc1_b, fc2_w, fc2_b):
    return pl.pallas_call(
        _ffn_body,
        grid=(E,),
        in_specs=[
            pl.BlockSpec((CAP, D), lambda e: (e, 0)),
            pl.BlockSpec((1, H, D), lambda e: (e, 0, 0)),
            pl.BlockSpec((1, 1, H), lambda e: (e, 0, 0)),
            pl.BlockSpec((1, H, D), lambda e: (e, 0, 0)),
            pl.BlockSpec((1, 1, D), lambda e: (e, 0, 0)),
        ],
        out_specs=pl.BlockSpec((CAP, D), lambda e: (e, 0)),
        out_shape=jax.ShapeDtypeStruct((_ROWS, D), jnp.float32),
    )(disp, fc1_w, fc1_b.reshape(E, 1, H), fc2_w, fc2_b.reshape(E, 1, D))


def _combine(y, code_f, wgt_f):
    mesh = plsc.VectorSubcoreMesh(core_axis_name="c", subcore_axis_name="s")

    @functools.partial(
        pl.kernel,
        mesh=mesh,
        compiler_params=pltpu.CompilerParams(needs_layout_passes=False),
        out_type=jax.ShapeDtypeStruct((T, D), jnp.float32),
        scratch_types=[
            pltpu.VMEM((_TPT * K,), jnp.int32),
            pltpu.VMEM((_TPT * K,), jnp.float32),
            pltpu.VMEM((_TPT,), jnp.int32),
            pltpu.VMEM((_TPT,), jnp.int32),
            pltpu.VMEM((_TCHUNK, D), jnp.float32),
            pltpu.VMEM((_TCHUNK, D), jnp.float32),
            pltpu.VMEM((_TCHUNK, D), jnp.float32),
            pltpu.VMEM((_TCHUNK, D), jnp.float32),
            pltpu.VMEM((_TCHUNK, D), jnp.float32),
            pltpu.VMEM((_TCHUNK, D), jnp.float32),
            pltpu.SemaphoreType.DMA,
            pltpu.SemaphoreType.DMA,
            pltpu.SemaphoreType.DMA,
            pltpu.SemaphoreType.DMA,
        ],
    )
    def k(y_hbm, code_hbm, wgt_hbm, out_hbm,
          ci_vm, wf_vm, d0_vm, d1_vm, ra0_vm, ra1_vm, rb0_vm, rb1_vm,
          oa_vm, ob_vm, ga_sem, gb_sem, oa_sem, ob_sem):
        wid = lax.axis_index("s") * 2 + lax.axis_index("c")
        tbase = wid * _TPT
        pltpu.sync_copy(code_hbm.at[pl.ds(tbase * K, _TPT * K)], ci_vm)
        pltpu.sync_copy(wgt_hbm.at[pl.ds(tbase * K, _TPT * K)], wf_vm)

        lanes = lax.iota(jnp.int32, 16)

        def dbody(q, carry2):
            tloc = (q * 16 + lanes) * K
            c0 = plsc.load_gather(ci_vm, [tloc])
            c1 = plsc.load_gather(ci_vm, [tloc + 1])
            i0 = lax.shift_right_logical(c0, _PBITS)
            p0 = jnp.minimum(jnp.bitwise_and(c0, _PMASK), CAP - 1)
            i1 = lax.shift_right_logical(c1, _PBITS)
            p1 = jnp.minimum(jnp.bitwise_and(c1, _PMASK), CAP - 1)
            d0_vm[pl.ds(q * 16, 16)] = i0 * CAP + p0
            d1_vm[pl.ds(q * 16, 16)] = i1 * CAP + p1
            return carry2

        lax.fori_loop(0, _TPT // 16, dbody, 0, unroll=4)

        rbufs = ((ra0_vm, ra1_vm), (rb0_vm, rb1_vm))
        gsems = (ga_sem, gb_sem)
        obufs = (oa_vm, ob_vm)
        osems = (oa_sem, ob_sem)
        gwaits = [None, None]
        owaits = [None, None]

        def start_gather(c):
            b = c % 2
            i0r = d0_vm.at[pl.ds(c * _TCHUNK, _TCHUNK)]
            i1r = d1_vm.at[pl.ds(c * _TCHUNK, _TCHUNK)]
            gwaits[b] = (
                pltpu.async_copy(y_hbm.at[i0r], rbufs[b][0], gsems[b]),
                pltpu.async_copy(y_hbm.at[i1r], rbufs[b][1], gsems[b]),
            )

        start_gather(0)
        for c in range(_NTC):
            b = c % 2
            if c + 1 < _NTC:
                start_gather(c + 1)
            for h in gwaits[b]:
                h.wait()
            if owaits[b] is not None:
                owaits[b].wait()
            o_vm = obufs[b]
            r0_vm, r1_vm = rbufs[b]

            def rbody(r, carry2):
                tl = (c * _TCHUNK + r) * K + jnp.zeros((16,), jnp.int32)
                w0 = plsc.load_gather(wf_vm, [tl])
                w1 = plsc.load_gather(wf_vm, [tl + 1])

                def cbody(cc, carry3):
                    a = r0_vm[r, pl.ds(cc * 16, 16)]
                    bb = r1_vm[r, pl.ds(cc * 16, 16)]
                    o_vm[r, pl.ds(cc * 16, 16)] = w0 * a + w1 * bb
                    return carry3

                lax.fori_loop(0, D // 16, cbody, 0, unroll=8)
                return carry2

            lax.fori_loop(0, _TCHUNK, rbody, 0)
            dst = out_hbm.at[pl.ds(tbase + c * _TCHUNK, _TCHUNK)]
            owaits[b] = pltpu.async_copy(o_vm, dst, osems[b])

        for w in owaits:
            if w is not None:
                w.wait()

    return k(y, code_f, wgt_f)


def kernel(x, wg, fc1_w, fc1_b, fc2_w, fc2_b):
    code, wgt = _router(x, wg)
    code_f = code.reshape(T * K)
    wgt_f = wgt.reshape(T * K)
    x_ext = jnp.concatenate([jnp.zeros((1, D), x.dtype), x], axis=0)
    disp = _dispatch(x_ext, code_f)
    y = _ffn(disp, fc1_w, fc1_b, fc2_w, fc2_b)
    return _combine(y, code_f, wgt_f)
